# Initial kernel scaffold; baseline (speedup 1.0000x reference)
import jax
import jax.numpy as jnp
from jax.experimental import pallas as pl


def kernel(input_ids, token_type_ids, word_table, pos_table, type_table, ln_gamma, ln_beta):
    raise NotImplementedError("write your pallas kernel here")


# speedup vs baseline: 1.2096x; 1.2096x over previous
import functools

import jax
import jax.numpy as jnp
from jax import lax
from jax.experimental import pallas as pl
from jax.experimental.pallas import tpu as pltpu
from jax.experimental.pallas import tpu_sc as plsc

_B, _S, _H = 64, 512, 768
_BS = _B * _S
_EPS = 1e-12
_LANES = 16
_CH = 64


def _sc_info():
    info = plsc.get_sparse_core_info()
    return info.num_cores, info.num_subcores


def _sc_gather_body(nc, tpw, nchunk, ids_hbm, tt_hbm, word_hbm, type_hbm,
                    out_hbm, idx_v, tt_v, wbuf, tbuf, sem_w, sem_t):
    wid = lax.axis_index("s") * nc + lax.axis_index("c")
    base = wid * tpw

    def chunk(c, carry):
        tok0 = base + c * _CH
        pltpu.sync_copy(ids_hbm.at[pl.ds(tok0, _CH)], idx_v)
        pltpu.sync_copy(tt_hbm.at[pl.ds(tok0, _CH)], tt_v)
        cp_w = pltpu.async_copy(word_hbm.at[idx_v], wbuf, sem_w)
        cp_t = pltpu.async_copy(type_hbm.at[tt_v], tbuf, sem_t)
        cp_w.wait()
        cp_t.wait()

        def tok(t, c2):
            for h in range(_H // _LANES):
                sl = pl.ds(h * _LANES, _LANES)
                wbuf[t, sl] = wbuf[t, sl] + tbuf[t, sl]
            return c2

        lax.fori_loop(0, _CH, tok, 0)
        pltpu.sync_copy(wbuf, out_hbm.at[pl.ds(tok0, _CH)])
        return carry

    lax.fori_loop(0, nchunk, chunk, 0)


def _make_sc_gather():
    nc, ns = _sc_info()
    nw = nc * ns
    tpw = _BS // nw
    mesh = plsc.VectorSubcoreMesh(core_axis_name="c", subcore_axis_name="s")
    return pl.kernel(
        functools.partial(_sc_gather_body, nc, tpw, tpw // _CH),
        mesh=mesh,
        out_type=jax.ShapeDtypeStruct((_BS, _H), jnp.float32),
        scratch_types=[
            pltpu.VMEM((_CH,), jnp.int32),
            pltpu.VMEM((_CH,), jnp.int32),
            pltpu.VMEM((_CH, _H), jnp.float32),
            pltpu.VMEM((_CH, _H), jnp.float32),
            pltpu.SemaphoreType.DMA,
            pltpu.SemaphoreType.DMA,
        ],
    )


def _tc_ln_body(x_ref, pos_ref, g_ref, b_ref, o_ref):
    x = x_ref[0] + pos_ref[...]
    mean = jnp.mean(x, axis=-1, keepdims=True)
    xc = x - mean
    var = jnp.mean(xc * xc, axis=-1, keepdims=True)
    inv = lax.rsqrt(var + _EPS)
    o_ref[0] = (xc * inv) * g_ref[0:1] + b_ref[0:1]


def _tc_ln(x, pos_table, gamma8, beta8):
    return pl.pallas_call(
        _tc_ln_body,
        grid=(_B,),
        in_specs=[
            pl.BlockSpec((1, _S, _H), lambda i: (i, 0, 0)),
            pl.BlockSpec((_S, _H), lambda i: (0, 0)),
            pl.BlockSpec((8, _H), lambda i: (0, 0)),
            pl.BlockSpec((8, _H), lambda i: (0, 0)),
        ],
        out_specs=pl.BlockSpec((1, _S, _H), lambda i: (i, 0, 0)),
        out_shape=jax.ShapeDtypeStruct((_B, _S, _H), jnp.float32),
    )(x, pos_table, gamma8, beta8)


def kernel(input_ids, token_type_ids, word_table, pos_table, type_table,
           ln_gamma, ln_beta):
    ids = input_ids.reshape(-1).astype(jnp.int32)
    tts = token_type_ids.reshape(-1).astype(jnp.int32)
    sc_gather = _make_sc_gather()
    summed = sc_gather(ids, tts, word_table, type_table)
    gamma8 = jnp.broadcast_to(ln_gamma[None, :], (8, _H))
    beta8 = jnp.broadcast_to(ln_beta[None, :], (8, _H))
    out = _tc_ln(summed.reshape(_B, _S, _H), pos_table, gamma8, beta8)
    mask = jnp.ones((_B, _S), dtype=jnp.int32)
    return (out, mask)


# speedup vs baseline: 6.0054x; 4.9648x over previous
import functools

import jax
import jax.numpy as jnp
from jax import lax
from jax.experimental import pallas as pl
from jax.experimental.pallas import tpu as pltpu
from jax.experimental.pallas import tpu_sc as plsc

_B, _S, _H = 64, 512, 768
_BS = _B * _S
_EPS = 1e-12
_CH = 64


def _sc_info():
    info = plsc.get_sparse_core_info()
    return info.num_cores, info.num_subcores


def _sc_gather_body(nc, tpw, nchunk, ids_hbm, word_hbm, out_hbm,
                    idx0, idx1, buf0, buf1, gsem0, gsem1, osem0, osem1):
    wid = lax.axis_index("s") * nc + lax.axis_index("c")
    base = wid * tpw
    idxs = (idx0, idx1)
    bufs = (buf0, buf1)
    gsems = (gsem0, gsem1)
    osems = (osem0, osem1)

    def stage_and_fire(c):
        k = c & 1
        tok0 = base + c * _CH
        pltpu.sync_copy(ids_hbm.at[pl.ds(tok0, _CH)], idxs[k])
        return pltpu.async_copy(word_hbm.at[idxs[k]], bufs[k], gsems[k])

    gather = {0: stage_and_fire(0)}
    store = {}
    for c in range(nchunk):
        k = c & 1
        if c + 1 < nchunk:
            if c - 1 >= 0:
                store[c - 1].wait()
            gather[c + 1] = stage_and_fire(c + 1)
        gather[c].wait()
        tok0 = base + c * _CH
        store[c] = pltpu.async_copy(bufs[k], out_hbm.at[pl.ds(tok0, _CH)],
                                    osems[k])
    store[nchunk - 2].wait()
    store[nchunk - 1].wait()


def _make_sc_gather():
    nc, ns = _sc_info()
    nw = nc * ns
    tpw = _BS // nw
    mesh = plsc.VectorSubcoreMesh(core_axis_name="c", subcore_axis_name="s")
    return pl.kernel(
        functools.partial(_sc_gather_body, nc, tpw, tpw // _CH),
        mesh=mesh,
        out_type=jax.ShapeDtypeStruct((_BS, _H), jnp.float32),
        scratch_types=[
            pltpu.VMEM((_CH,), jnp.int32),
            pltpu.VMEM((_CH,), jnp.int32),
            pltpu.VMEM((_CH, _H), jnp.float32),
            pltpu.VMEM((_CH, _H), jnp.float32),
            pltpu.SemaphoreType.DMA,
            pltpu.SemaphoreType.DMA,
            pltpu.SemaphoreType.DMA,
            pltpu.SemaphoreType.DMA,
        ],
    )


def _tc_ln_body(x_ref, pos_ref, tt_ref, t0_ref, d_ref, g_ref, b_ref, o_ref):
    ttf = tt_ref[0].astype(jnp.float32)
    ttcol = jnp.transpose(ttf)
    x = x_ref[0] + pos_ref[...] + t0_ref[0:1] + ttcol * d_ref[0:1]
    mean = jnp.mean(x, axis=-1, keepdims=True)
    xc = x - mean
    var = jnp.mean(xc * xc, axis=-1, keepdims=True)
    inv = lax.rsqrt(var + _EPS)
    o_ref[0] = (xc * inv) * g_ref[0:1] + b_ref[0:1]


def _tc_ln(x, pos_table, tt3, t08, d8, gamma8, beta8):
    return pl.pallas_call(
        _tc_ln_body,
        grid=(_B,),
        in_specs=[
            pl.BlockSpec((1, _S, _H), lambda i: (i, 0, 0)),
            pl.BlockSpec((_S, _H), lambda i: (0, 0)),
            pl.BlockSpec((1, 1, _S), lambda i: (i, 0, 0)),
            pl.BlockSpec((8, _H), lambda i: (0, 0)),
            pl.BlockSpec((8, _H), lambda i: (0, 0)),
            pl.BlockSpec((8, _H), lambda i: (0, 0)),
            pl.BlockSpec((8, _H), lambda i: (0, 0)),
        ],
        out_specs=pl.BlockSpec((1, _S, _H), lambda i: (i, 0, 0)),
        out_shape=jax.ShapeDtypeStruct((_B, _S, _H), jnp.float32),
    )(x, pos_table, tt3, t08, d8, gamma8, beta8)


def kernel(input_ids, token_type_ids, word_table, pos_table, type_table,
           ln_gamma, ln_beta):
    ids = input_ids.reshape(-1).astype(jnp.int32)
    tt3 = token_type_ids.reshape(_B, 1, _S).astype(jnp.int32)
    sc_gather = _make_sc_gather()
    words = sc_gather(ids, word_table)
    t0 = type_table[0]
    d = type_table[1] - t0
    t08 = jnp.broadcast_to(t0[None, :], (8, _H))
    d8 = jnp.broadcast_to(d[None, :], (8, _H))
    gamma8 = jnp.broadcast_to(ln_gamma[None, :], (8, _H))
    beta8 = jnp.broadcast_to(ln_beta[None, :], (8, _H))
    out = _tc_ln(words.reshape(_B, _S, _H), pos_table, tt3, t08, d8,
                 gamma8, beta8)
    mask = jnp.ones((_B, _S), dtype=jnp.int32)
    return (out, mask)
